# Initial kernel scaffold; baseline (speedup 1.0000x reference)
import jax
import jax.numpy as jnp
from jax.experimental import pallas as pl


def kernel(x, adj, W0, W1, b):
    raise NotImplementedError("write your pallas kernel here")


# speedup vs baseline: 16.5029x; 16.5029x over previous
import functools

import jax
import jax.numpy as jnp
from jax import lax
from jax.experimental import pallas as pl
from jax.experimental.pallas import tpu as pltpu
from jax.experimental.pallas import tpu_sc as plsc

N = 10000
E = 320000
F = 128

NC = 2
NS = 16
NW = NC * NS

CHUNK = 128
NCHUNKS = E // CHUNK
NPAD = 10112
DEG_SLICE = NPAD // NS
ROWS_T = NPAD // NS

_mesh = plsc.VectorSubcoreMesh(core_axis_name="c", subcore_axis_name="s")


@functools.partial(
    pl.kernel,
    out_type=jax.ShapeDtypeStruct((NPAD,), jnp.float32),
    mesh=_mesh,
    scratch_types=[
        pltpu.VMEM((CHUNK,), jnp.int32),
        pltpu.VMEM((CHUNK,), jnp.float32),
        pltpu.VMEM((DEG_SLICE,), jnp.float32),
        pltpu.VMEM_SHARED((NPAD,), jnp.float32),
    ],
)
def _deg_kernel(row_hbm, zeros_hbm, ones_hbm, deg_hbm, idx_v, ones_v, zv, acc):
    c = lax.axis_index("c")
    s = lax.axis_index("s")
    pltpu.sync_copy(zeros_hbm, zv)
    pltpu.sync_copy(zv, acc.at[pl.ds(s * DEG_SLICE, DEG_SLICE)])
    pltpu.sync_copy(ones_hbm, ones_v)
    plsc.subcore_barrier()

    nchunks_t = 156 + jnp.where(s < 4, 1, 0)

    def body(g, _):
        ci = s + g * NS
        pltpu.sync_copy(row_hbm.at[pl.ds(ci * CHUNK, CHUNK)], idx_v)
        pltpu.sync_copy(ones_v, acc.at[idx_v], add=True)
        return 0

    lax.fori_loop(0, nchunks_t, body, 0)
    plsc.subcore_barrier()

    @pl.when(c == 0)
    def _():
        pltpu.sync_copy(acc.at[pl.ds(s * DEG_SLICE, DEG_SLICE)], zv)
        pltpu.sync_copy(zv, deg_hbm.at[pl.ds(s * DEG_SLICE, DEG_SLICE)])


@functools.partial(
    pl.kernel,
    out_type=jax.ShapeDtypeStruct((NC, NPAD, F), jnp.float32),
    mesh=_mesh,
    scratch_types=[
        pltpu.VMEM((CHUNK,), jnp.int32),
        pltpu.VMEM((CHUNK,), jnp.int32),
        pltpu.VMEM((CHUNK, F), jnp.float32),
        pltpu.VMEM_SHARED((NPAD, F), jnp.float32),
        pltpu.SemaphoreType.DMA,
    ],
)
def _seg_kernel(xs_hbm, row_hbm, col_hbm, zblk_hbm, s_hbm,
                rowi_v, coli_v, rows_v, acc, gsem):
    c = lax.axis_index("c")
    s = lax.axis_index("s")
    w = c * NS + s

    pltpu.sync_copy(zblk_hbm, rows_v)
    for j in range(4):
        pltpu.sync_copy(rows_v,
                        acc.at[pl.ds(s * ROWS_T + j * CHUNK, CHUNK)])
    pltpu.sync_copy(rows_v.at[pl.ds(0, 120)],
                    acc.at[pl.ds(s * ROWS_T + 4 * CHUNK, 120)])
    plsc.subcore_barrier()

    nchunks_w = 78 + jnp.where(w < 4, 1, 0)

    def body(g, _):
        base = (w + g * NW) * CHUNK
        pltpu.sync_copy(row_hbm.at[pl.ds(base, CHUNK)], rowi_v)
        pltpu.sync_copy(col_hbm.at[pl.ds(base, CHUNK)], coli_v)
        pltpu.async_copy(xs_hbm.at[rowi_v], rows_v, gsem).wait()
        pltpu.sync_copy(rows_v, acc.at[coli_v], add=True)
        return 0

    lax.fori_loop(0, nchunks_w, body, 0)
    plsc.subcore_barrier()

    pltpu.sync_copy(acc.at[pl.ds(s * ROWS_T, ROWS_T)],
                    s_hbm.at[c, pl.ds(s * ROWS_T, ROWS_T)])


_RB = 400


def _k2_body(x_ref, deg_ref, w0_ref, w1_ref, b_ref, xs_ref, z0_ref, dinv_ref):
    x = x_ref[...]
    deg = deg_ref[...]
    dinv = jnp.where(deg > 0, lax.rsqrt(deg), 0.0)
    xs_ref[...] = dinv * jnp.dot(x, w1_ref[...], preferred_element_type=jnp.float32)
    z0_ref[...] = jnp.dot(x, w0_ref[...], preferred_element_type=jnp.float32) + b_ref[...]
    dinv_ref[...] = dinv


def _k4_body(z0_ref, dinv_ref, s_ref, o_ref):
    stot = s_ref[0] + s_ref[1]
    o_ref[...] = jnp.maximum(z0_ref[...] - dinv_ref[...] * stot, 0.0)


def kernel(x, adj, W0, W1, b):
    row = adj[0]
    col = adj[1]
    zeros_deg = jnp.zeros((DEG_SLICE,), jnp.float32)
    ones_chunk = jnp.ones((CHUNK,), jnp.float32)
    zblk = jnp.zeros((CHUNK, F), jnp.float32)

    deg = _deg_kernel(row, zeros_deg, ones_chunk)
    deg2 = deg[:N, None]

    xs, z0, dinv = pl.pallas_call(
        _k2_body,
        grid=(N // _RB,),
        in_specs=[
            pl.BlockSpec((_RB, F), lambda i: (i, 0)),
            pl.BlockSpec((_RB, 1), lambda i: (i, 0)),
            pl.BlockSpec((F, F), lambda i: (0, 0)),
            pl.BlockSpec((F, F), lambda i: (0, 0)),
            pl.BlockSpec((1, F), lambda i: (0, 0)),
        ],
        out_specs=[
            pl.BlockSpec((_RB, F), lambda i: (i, 0)),
            pl.BlockSpec((_RB, F), lambda i: (i, 0)),
            pl.BlockSpec((_RB, 1), lambda i: (i, 0)),
        ],
        out_shape=[
            jax.ShapeDtypeStruct((N, F), jnp.float32),
            jax.ShapeDtypeStruct((N, F), jnp.float32),
            jax.ShapeDtypeStruct((N, 1), jnp.float32),
        ],
    )(x, deg2, W0, W1, b.reshape(1, F))

    S = _seg_kernel(xs, row, col, zblk)

    out = pl.pallas_call(
        _k4_body,
        grid=(N // _RB,),
        in_specs=[
            pl.BlockSpec((_RB, F), lambda i: (i, 0)),
            pl.BlockSpec((_RB, 1), lambda i: (i, 0)),
            pl.BlockSpec((NC, _RB, F), lambda i: (0, i, 0)),
        ],
        out_specs=pl.BlockSpec((_RB, F), lambda i: (i, 0)),
        out_shape=jax.ShapeDtypeStruct((N, F), jnp.float32),
    )(z0, dinv, S)
    return out


# speedup vs baseline: 29.1838x; 1.7684x over previous
import functools

import jax
import jax.numpy as jnp
from jax import lax
from jax.experimental import pallas as pl
from jax.experimental.pallas import tpu as pltpu
from jax.experimental.pallas import tpu_sc as plsc

N = 10000
E = 320000
F = 128

NC = 2
NS = 16
NW = NC * NS

CHUNK = 128
NPAD = 10112
DEG_SLICE = NPAD // NS
ROWS_T = NPAD // NS
GPW = 80
NCHP = 2624
E_PAD = NCHP * CHUNK

_mesh = plsc.VectorSubcoreMesh(core_axis_name="c", subcore_axis_name="s")


@functools.partial(
    pl.kernel,
    out_type=jax.ShapeDtypeStruct((NC * NPAD,), jnp.float32),
    mesh=_mesh,
    scratch_types=[
        pltpu.VMEM((CHUNK,), jnp.int32),
        pltpu.VMEM((CHUNK,), jnp.int32),
        pltpu.VMEM((CHUNK,), jnp.float32),
        pltpu.VMEM((DEG_SLICE,), jnp.float32),
        pltpu.VMEM_SHARED((NPAD,), jnp.float32),
        pltpu.SemaphoreType.DMA,
        pltpu.SemaphoreType.DMA,
    ],
)
def _deg_kernel(row_hbm, zeros_hbm, ones_hbm, deg_hbm,
                i0, i1, ones_v, zv, acc, s0, s1):
    c = lax.axis_index("c")
    s = lax.axis_index("s")
    w = c * NS + s
    I = (i0, i1)
    SEM = (s0, s1)

    def start_idx(g, p):
        pltpu.async_copy(row_hbm.at[pl.ds((w + g * NW) * CHUNK, CHUNK)],
                         I[p], SEM[p])

    def wait_idx(g, p):
        pltpu.make_async_copy(row_hbm.at[pl.ds((w + g * NW) * CHUNK, CHUNK)],
                              I[p], SEM[p]).wait()

    pltpu.sync_copy(zeros_hbm, zv)
    pltpu.sync_copy(zv, acc.at[pl.ds(s * DEG_SLICE, DEG_SLICE)])
    pltpu.sync_copy(ones_hbm, ones_v)
    plsc.subcore_barrier()

    start_idx(0, 0)

    def body(j, _):
        g = j * 2
        for p in range(2):
            wait_idx(g + p, p)
            start_idx(g + p + 1, 1 - p)
            pltpu.sync_copy(ones_v, acc.at[I[p]], add=True)
        return 0

    lax.fori_loop(0, GPW // 2, body, 0)
    wait_idx(GPW, 0)
    plsc.subcore_barrier()

    pltpu.sync_copy(acc.at[pl.ds(s * DEG_SLICE, DEG_SLICE)], zv)
    pltpu.sync_copy(zv, deg_hbm.at[pl.ds(c * NPAD + s * DEG_SLICE, DEG_SLICE)])


@functools.partial(
    pl.kernel,
    out_type=jax.ShapeDtypeStruct((NC, NPAD, F), jnp.float32),
    mesh=_mesh,
    scratch_types=[
        pltpu.VMEM((CHUNK,), jnp.int32),
        pltpu.VMEM((CHUNK,), jnp.int32),
        pltpu.VMEM((CHUNK,), jnp.int32),
        pltpu.VMEM((CHUNK,), jnp.int32),
        pltpu.VMEM((CHUNK, F), jnp.float32),
        pltpu.VMEM((CHUNK, F), jnp.float32),
        pltpu.VMEM_SHARED((NPAD, F), jnp.float32),
        pltpu.SemaphoreType.DMA,
        pltpu.SemaphoreType.DMA,
        pltpu.SemaphoreType.DMA,
        pltpu.SemaphoreType.DMA,
        pltpu.SemaphoreType.DMA,
        pltpu.SemaphoreType.DMA,
    ],
)
def _seg_kernel(xs_hbm, row_hbm, col_hbm, zblk_hbm, s_hbm,
                ri0, ri1, ci0, ci1, rv0, rv1, acc,
                sr0, sr1, sc0, sc1, sg0, sg1):
    c = lax.axis_index("c")
    s = lax.axis_index("s")
    w = c * NS + s
    RI = (ri0, ri1)
    CI = (ci0, ci1)
    RV = (rv0, rv1)
    SR = (sr0, sr1)
    SC = (sc0, sc1)
    SG = (sg0, sg1)

    def base(g):
        return (w + g * NW) * CHUNK

    def start_row_idx(g, p):
        pltpu.async_copy(row_hbm.at[pl.ds(base(g), CHUNK)], RI[p], SR[p])

    def wait_row_idx(g, p):
        pltpu.make_async_copy(row_hbm.at[pl.ds(base(g), CHUNK)], RI[p], SR[p]).wait()

    def start_col_idx(g, p):
        pltpu.async_copy(col_hbm.at[pl.ds(base(g), CHUNK)], CI[p], SC[p])

    def wait_col_idx(g, p):
        pltpu.make_async_copy(col_hbm.at[pl.ds(base(g), CHUNK)], CI[p], SC[p]).wait()

    def start_gather(p):
        pltpu.async_copy(xs_hbm.at[RI[p]], RV[p], SG[p])

    def wait_gather(p):
        pltpu.make_async_copy(xs_hbm.at[RI[p]], RV[p], SG[p]).wait()

    pltpu.sync_copy(zblk_hbm, rv0)
    for j in range(4):
        pltpu.sync_copy(rv0, acc.at[pl.ds(s * ROWS_T + j * CHUNK, CHUNK)])
    pltpu.sync_copy(rv0.at[pl.ds(0, 120)],
                    acc.at[pl.ds(s * ROWS_T + 4 * CHUNK, 120)])
    plsc.subcore_barrier()

    start_row_idx(0, 0)
    start_col_idx(0, 0)
    wait_row_idx(0, 0)
    start_gather(0)
    start_row_idx(1, 1)
    start_col_idx(1, 1)

    def body(j, _):
        g0 = j * 2
        for p in range(2):
            g = g0 + p
            wait_row_idx(g + 1, 1 - p)
            start_gather(1 - p)
            wait_gather(p)
            start_row_idx(g + 2, p)
            wait_col_idx(g, p)
            pltpu.sync_copy(RV[p], acc.at[CI[p]], add=True)
            start_col_idx(g + 2, p)
        return 0

    lax.fori_loop(0, GPW // 2, body, 0)

    wait_row_idx(GPW + 1, 1)
    wait_col_idx(GPW + 1, 1)
    wait_gather(0)
    wait_col_idx(GPW, 0)
    plsc.subcore_barrier()

    pltpu.sync_copy(acc.at[pl.ds(s * ROWS_T, ROWS_T)],
                    s_hbm.at[c, pl.ds(s * ROWS_T, ROWS_T)])


_RB = 400


def _k2_body(x_ref, dega_ref, degb_ref, w0_ref, w1_ref, b_ref,
             xs_ref, z0_ref, dinv_ref):
    x = x_ref[...]
    deg = dega_ref[...] + degb_ref[...]
    dinv = jnp.where(deg > 0, lax.rsqrt(deg), 0.0)
    xs_ref[...] = dinv * jnp.dot(x, w1_ref[...], preferred_element_type=jnp.float32)
    z0_ref[...] = jnp.dot(x, w0_ref[...], preferred_element_type=jnp.float32) + b_ref[...]
    dinv_ref[...] = dinv


def _k4_body(z0_ref, dinv_ref, s_ref, o_ref):
    stot = s_ref[0] + s_ref[1]
    o_ref[...] = jnp.maximum(z0_ref[...] - dinv_ref[...] * stot, 0.0)


def kernel(x, adj, W0, W1, b):
    row = adj[0]
    col = adj[1]
    pad_i = jnp.arange(E_PAD - E, dtype=jnp.int32)
    dummy = N + pad_i % (NPAD - N)
    row_g = jnp.concatenate([row, pad_i % N])
    col_s = jnp.concatenate([col, dummy])
    row_d = jnp.concatenate([row, dummy])
    zeros_deg = jnp.zeros((DEG_SLICE,), jnp.float32)
    ones_chunk = jnp.ones((CHUNK,), jnp.float32)
    zblk = jnp.zeros((CHUNK, F), jnp.float32)

    deg = _deg_kernel(row_d, zeros_deg, ones_chunk)
    dega = deg[:N, None]
    degb = deg[NPAD:NPAD + N, None]

    xs, z0, dinv = pl.pallas_call(
        _k2_body,
        grid=(N // _RB,),
        in_specs=[
            pl.BlockSpec((_RB, F), lambda i: (i, 0)),
            pl.BlockSpec((_RB, 1), lambda i: (i, 0)),
            pl.BlockSpec((_RB, 1), lambda i: (i, 0)),
            pl.BlockSpec((F, F), lambda i: (0, 0)),
            pl.BlockSpec((F, F), lambda i: (0, 0)),
            pl.BlockSpec((1, F), lambda i: (0, 0)),
        ],
        out_specs=[
            pl.BlockSpec((_RB, F), lambda i: (i, 0)),
            pl.BlockSpec((_RB, F), lambda i: (i, 0)),
            pl.BlockSpec((_RB, 1), lambda i: (i, 0)),
        ],
        out_shape=[
            jax.ShapeDtypeStruct((N, F), jnp.float32),
            jax.ShapeDtypeStruct((N, F), jnp.float32),
            jax.ShapeDtypeStruct((N, 1), jnp.float32),
        ],
    )(x, dega, degb, W0, W1, b.reshape(1, F))

    S = _seg_kernel(xs, row_g, col_s, zblk)

    out = pl.pallas_call(
        _k4_body,
        grid=(N // _RB,),
        in_specs=[
            pl.BlockSpec((_RB, F), lambda i: (i, 0)),
            pl.BlockSpec((_RB, 1), lambda i: (i, 0)),
            pl.BlockSpec((NC, _RB, F), lambda i: (0, i, 0)),
        ],
        out_specs=pl.BlockSpec((_RB, F), lambda i: (i, 0)),
        out_shape=jax.ShapeDtypeStruct((N, F), jnp.float32),
    )(z0, dinv, S)
    return out
